# baseline (speedup 1.0000x reference)
import functools

import jax
import jax.numpy as jnp
from jax import lax
from jax.experimental import pallas as pl
from jax.experimental.pallas import tpu as pltpu
from jax.experimental.pallas import tpu_sc as plsc

D = 128
N_NODES = 10000
N_EDGES = 320000

NC = 2
NS = 16
NW = NC * NS

G_CHUNK = 128
E_PAD = 327680

N_PAD = 10240
ROWS_PER_TILE = N_PAD // NS

N_HALF = N_PAD // 2
S_ACC = N_HALF + 128
S_PER_TILE = N_EDGES // NS
S_FULL = S_PER_TILE // 128
S_TAIL = S_PER_TILE - S_FULL * 128

BE = 2000
BN = 1000

_SC_MESH = plsc.VectorSubcoreMesh(
    core_axis_name="c", subcore_axis_name="s", num_cores=NC, num_subcores=NS
)


G_NBUF = 2
G_PER_TILE = E_PAD // NS // G_CHUNK


def _gather_body(a_hbm, b_hbm, src_hbm, dst_hbm, ga_hbm, gb_hbm,
                 table_sh, idx0, idx1, r0, r1,
                 gsem0, gsem1, ssem0, ssem1):
    c = lax.axis_index("c")
    s = lax.axis_index("s")
    idxb = (idx0, idx1)
    rows = (r0, r1)
    gsem = (gsem0, gsem1)
    ssem = (ssem0, ssem1)

    stg = pl.ds(s * ROWS_PER_TILE, ROWS_PER_TILE)

    @pl.when(c == 0)
    def _():
        pltpu.sync_copy(a_hbm.at[stg], table_sh.at[stg])

    @pl.when(c == 1)
    def _():
        pltpu.sync_copy(b_hbm.at[stg], table_sh.at[stg])

    plsc.subcore_barrier()

    base = s * (G_PER_TILE * G_CHUNK)

    def load_idx(p, off):
        @pl.when(c == 0)
        def _():
            pltpu.sync_copy(src_hbm.at[pl.ds(off, G_CHUNK)], idxb[p])

        @pl.when(c == 1)
        def _():
            pltpu.sync_copy(dst_hbm.at[pl.ds(off, G_CHUNK)], idxb[p])

    def fire_gather(p, off):
        load_idx(p, off)
        pltpu.async_copy(table_sh.at[idxb[p]], rows[p], gsem[p])

    def fire_store(p, off):
        @pl.when(c == 0)
        def _():
            pltpu.async_copy(rows[p], ga_hbm.at[pl.ds(off, G_CHUNK)], ssem[p])

        @pl.when(c == 1)
        def _():
            pltpu.async_copy(rows[p], gb_hbm.at[pl.ds(off, G_CHUNK)], ssem[p])

    for p in range(G_NBUF):
        fire_gather(p, base + p * G_CHUNK)

    def drain_store(p, off):
        pltpu.make_async_copy(rows[p], ga_hbm.at[pl.ds(off, G_CHUNK)],
                              ssem[p]).wait()

    def body(g, carry):
        for p in range(G_NBUF):
            cidx = g * G_NBUF + p
            off = base + cidx * G_CHUNK
            pltpu.make_async_copy(table_sh.at[idxb[p]], rows[p],
                                  gsem[p]).wait()
            fire_store(p, off)

            @pl.when(cidx + G_NBUF < G_PER_TILE)
            def _():
                drain_store(p, off)
                fire_gather(p, off + G_NBUF * G_CHUNK)

        return carry

    lax.fori_loop(0, G_PER_TILE // G_NBUF, body, 0)
    for p in range(G_NBUF):
        drain_store(p, base)


_gather_call = pl.kernel(
    _gather_body,
    out_type=[
        jax.ShapeDtypeStruct((E_PAD, D), jnp.float32),
        jax.ShapeDtypeStruct((E_PAD, D), jnp.float32),
    ],
    mesh=_SC_MESH,
    scratch_types=[
        pltpu.VMEM_SHARED((N_PAD, D), jnp.float32),
        pltpu.VMEM((G_CHUNK,), jnp.int32),
        pltpu.VMEM((G_CHUNK,), jnp.int32),
        pltpu.VMEM((G_CHUNK, D), jnp.float32),
        pltpu.VMEM((G_CHUNK, D), jnp.float32),
        pltpu.SemaphoreType.DMA,
        pltpu.SemaphoreType.DMA,
        pltpu.SemaphoreType.DMA,
        pltpu.SemaphoreType.DMA,
    ],
)


def _scatter_body(ueh_hbm, srcl_hbm, dstl_hbm, srch_hbm, dsth_hbm, zero_hbm,
                  out0_hbm, out1_hbm,
                  acc_sh, idxs0, idxs1, idxd0, idxd1, rows0, rows1,
                  idx_t, rows_t, lsem0, lsem1):
    c = lax.axis_index("c")
    s = lax.axis_index("s")
    arows = S_ACC // NS
    zbase = s * arows
    pltpu.sync_copy(zero_hbm.at[pl.ds(zbase, arows)],
                    acc_sh.at[pl.ds(zbase, arows)])
    plsc.subcore_barrier()

    base = s * S_PER_TILE
    idxs = (idxs0, idxs1)
    idxd = (idxd0, idxd1)
    rows = (rows0, rows1)
    lsem = (lsem0, lsem1)

    def fire_loads(p, off):
        pltpu.async_copy(ueh_hbm.at[pl.ds(off, 128)], rows[p], lsem[p])

        @pl.when(c == 0)
        def _():
            pltpu.async_copy(srcl_hbm.at[pl.ds(off, 128)], idxs[p], lsem[p])
            pltpu.async_copy(dstl_hbm.at[pl.ds(off, 128)], idxd[p], lsem[p])

        @pl.when(c == 1)
        def _():
            pltpu.async_copy(srch_hbm.at[pl.ds(off, 128)], idxs[p], lsem[p])
            pltpu.async_copy(dsth_hbm.at[pl.ds(off, 128)], idxd[p], lsem[p])

    def wait_loads(p, off):
        pltpu.make_async_copy(ueh_hbm.at[pl.ds(off, 128)], rows[p],
                              lsem[p]).wait()
        pltpu.make_async_copy(srcl_hbm.at[pl.ds(off, 128)], idxs[p],
                              lsem[p]).wait()
        pltpu.make_async_copy(dstl_hbm.at[pl.ds(off, 128)], idxd[p],
                              lsem[p]).wait()

    fire_loads(0, base)

    def body(g, carry):
        for p in range(2):
            i = g * 2 + p
            off = base + i * 128

            @pl.when(i + 1 < S_FULL)
            def _():
                fire_loads(1 - p, off + 128)

            wait_loads(p, off)
            pltpu.sync_copy(rows[p], acc_sh.at[idxs[p]], add=True)
            pltpu.sync_copy(rows[p], acc_sh.at[idxd[p]], add=True)
        return carry

    lax.fori_loop(0, S_FULL // 2, body, 0)

    toff = base + S_FULL * 128
    pltpu.sync_copy(ueh_hbm.at[pl.ds(toff, S_TAIL)], rows_t)

    @pl.when(c == 0)
    def _():
        pltpu.sync_copy(srcl_hbm.at[pl.ds(toff, S_TAIL)], idx_t)

    @pl.when(c == 1)
    def _():
        pltpu.sync_copy(srch_hbm.at[pl.ds(toff, S_TAIL)], idx_t)

    pltpu.sync_copy(rows_t, acc_sh.at[idx_t], add=True)

    @pl.when(c == 0)
    def _():
        pltpu.sync_copy(dstl_hbm.at[pl.ds(toff, S_TAIL)], idx_t)

    @pl.when(c == 1)
    def _():
        pltpu.sync_copy(dsth_hbm.at[pl.ds(toff, S_TAIL)], idx_t)

    pltpu.sync_copy(rows_t, acc_sh.at[idx_t], add=True)

    plsc.subcore_barrier()

    @pl.when(c == 0)
    def _():
        pltpu.sync_copy(acc_sh.at[pl.ds(zbase, arows)],
                        out0_hbm.at[pl.ds(zbase, arows)])

    @pl.when(c == 1)
    def _():
        pltpu.sync_copy(acc_sh.at[pl.ds(zbase, arows)],
                        out1_hbm.at[pl.ds(zbase, arows)])


_scatter_call = pl.kernel(
    _scatter_body,
    out_type=[
        jax.ShapeDtypeStruct((S_ACC, D), jnp.float32),
        jax.ShapeDtypeStruct((S_ACC, D), jnp.float32),
    ],
    mesh=_SC_MESH,
    scratch_types=[
        pltpu.VMEM_SHARED((S_ACC, D), jnp.float32),
        pltpu.VMEM((128,), jnp.int32),
        pltpu.VMEM((128,), jnp.int32),
        pltpu.VMEM((128,), jnp.int32),
        pltpu.VMEM((128,), jnp.int32),
        pltpu.VMEM((128, D), jnp.float32),
        pltpu.VMEM((128, D), jnp.float32),
        pltpu.VMEM((S_TAIL,), jnp.int32),
        pltpu.VMEM((S_TAIL, D), jnp.float32),
        pltpu.SemaphoreType.DMA,
        pltpu.SemaphoreType.DMA,
    ],
)


def _q_body(qemb, qw, qb, w1q0, eb10, w1q1, eb11, qb0_o, qb1_o):
    q = jnp.dot(qemb[...], qw[...], preferred_element_type=jnp.float32) + qb[...]
    qb0_o[...] = jnp.dot(q, w1q0[...], preferred_element_type=jnp.float32) + eb10[...]
    qb1_o[...] = jnp.dot(q, w1q1[...], preferred_element_type=jnp.float32) + eb11[...]


def _q_call(qemb, qw, qb, w1q0, eb10, w1q1, eb11):
    return pl.pallas_call(
        _q_body,
        out_shape=[
            jax.ShapeDtypeStruct((1, D), jnp.float32),
            jax.ShapeDtypeStruct((1, D), jnp.float32),
        ],
    )(qemb, qw, qb, w1q0, eb10, w1q1, eb11)


BNP = N_PAD // 10


def _prep_body(nh, w1s, w1d, a_o, b_o):
    nhv = nh[...]
    a_o[...] = jnp.dot(nhv, w1s[...], preferred_element_type=jnp.float32)
    b_o[...] = jnp.dot(nhv, w1d[...], preferred_element_type=jnp.float32)


def _prep_call(nh, w1s, w1d):
    nblk = pl.BlockSpec((BNP, D), lambda i: (i, 0))
    wblk = pl.BlockSpec((D, D), lambda i: (0, 0))
    return pl.pallas_call(
        _prep_body,
        grid=(N_PAD // BNP,),
        in_specs=[nblk, wblk, wblk],
        out_specs=[nblk, nblk],
        out_shape=[
            jax.ShapeDtypeStruct((N_PAD, D), jnp.float32),
            jax.ShapeDtypeStruct((N_PAD, D), jnp.float32),
        ],
        compiler_params=pltpu.CompilerParams(
            dimension_semantics=("parallel",)),
    )(nh, w1s, w1d)


def _edge_body(ga, gb, eh, w1e, w2, qb, eb2, ueh_o):
    x = ga[...] + gb[...] + qb[...]
    x = x + jnp.dot(eh[...], w1e[...], preferred_element_type=jnp.float32)
    h = jnp.maximum(x, 0.0)
    ueh_o[...] = jnp.dot(h, w2[...], preferred_element_type=jnp.float32) + eb2[...]


def _edge_score_body(ga, gb, eh, w1e, w2, qb, eb2, esw, esb, ueh_o, es_o):
    x = ga[...] + gb[...] + qb[...]
    x = x + jnp.dot(eh[...], w1e[...], preferred_element_type=jnp.float32)
    h = jnp.maximum(x, 0.0)
    ueh = jnp.dot(h, w2[...], preferred_element_type=jnp.float32) + eb2[...]
    ueh_o[...] = ueh
    es_o[...] = jax.nn.sigmoid(
        jnp.dot(ueh, esw[...], preferred_element_type=jnp.float32) + esb[...])


def _edge_call(ga, gb, eh, w1e, w2, qb, eb2):
    eblk = pl.BlockSpec((BE, D), lambda i: (i, 0))
    wblk = pl.BlockSpec((D, D), lambda i: (0, 0))
    vblk = pl.BlockSpec((1, D), lambda i: (0, 0))
    return pl.pallas_call(
        _edge_body,
        grid=(N_EDGES // BE,),
        in_specs=[eblk, eblk, eblk, wblk, wblk, vblk, vblk],
        out_specs=eblk,
        out_shape=jax.ShapeDtypeStruct((N_EDGES, D), jnp.float32),
        compiler_params=pltpu.CompilerParams(
            dimension_semantics=("parallel",)),
    )(ga, gb, eh, w1e, w2, qb, eb2)


def _edge_score_call(ga, gb, eh, w1e, w2, qb, eb2, esw, esb):
    eblk = pl.BlockSpec((BE, D), lambda i: (i, 0))
    wblk = pl.BlockSpec((D, D), lambda i: (0, 0))
    vblk = pl.BlockSpec((1, D), lambda i: (0, 0))
    return pl.pallas_call(
        _edge_score_body,
        grid=(N_EDGES // BE,),
        in_specs=[eblk, eblk, eblk, wblk, wblk, vblk, vblk,
                  pl.BlockSpec((D, 1), lambda i: (0, 0)),
                  pl.BlockSpec((1, 1), lambda i: (0, 0))],
        out_specs=[eblk, pl.BlockSpec((BE, 1), lambda i: (i, 0))],
        out_shape=[
            jax.ShapeDtypeStruct((N_EDGES, D), jnp.float32),
            jax.ShapeDtypeStruct((N_EDGES, 1), jnp.float32),
        ],
        compiler_params=pltpu.CompilerParams(
            dimension_semantics=("parallel",)),
    )(ga, gb, eh, w1e, w2, qb, eb2, esw, esb)


def _node_prep_body(nh, p0, p1, nw1a, nw1b, nb1, nw2, nb2, w1s, w1d,
                    nh_o, a_o, b_o):
    nhv = nh[...]
    agg = jnp.where(pl.program_id(0) < 5, p0[...], p1[...])
    h = jnp.dot(nhv, nw1a[...], preferred_element_type=jnp.float32)
    h = h + jnp.dot(agg, nw1b[...], preferred_element_type=jnp.float32)
    h = jnp.maximum(h + nb1[...], 0.0)
    nh2 = jnp.dot(h, nw2[...], preferred_element_type=jnp.float32) + nb2[...]
    nh_o[...] = nh2
    a_o[...] = jnp.dot(nh2, w1s[...], preferred_element_type=jnp.float32)
    b_o[...] = jnp.dot(nh2, w1d[...], preferred_element_type=jnp.float32)


def _node_prep_call(nh, p0, p1, nw1a, nw1b, nb1, nw2, nb2, w1s, w1d):
    nblk = pl.BlockSpec((BNP, D), lambda i: (i, 0))
    loblk = pl.BlockSpec((BNP, D), lambda i: (jnp.minimum(i, 4), 0))
    hiblk = pl.BlockSpec((BNP, D), lambda i: (jnp.maximum(i, 5) - 5, 0))
    wblk = pl.BlockSpec((D, D), lambda i: (0, 0))
    vblk = pl.BlockSpec((1, D), lambda i: (0, 0))
    return pl.pallas_call(
        _node_prep_body,
        grid=(N_PAD // BNP,),
        in_specs=[nblk, loblk, hiblk, wblk, wblk, vblk, wblk, vblk, wblk,
                  wblk],
        out_specs=[nblk, nblk, nblk],
        out_shape=[
            jax.ShapeDtypeStruct((N_PAD, D), jnp.float32),
            jax.ShapeDtypeStruct((N_PAD, D), jnp.float32),
            jax.ShapeDtypeStruct((N_PAD, D), jnp.float32),
        ],
        compiler_params=pltpu.CompilerParams(
            dimension_semantics=("parallel",)),
    )(nh, p0, p1, nw1a, nw1b, nb1, nw2, nb2, w1s, w1d)


def _node_final_body(nh, p0, p1, nw1a, nw1b, nb1, nw2, nb2, nsw, nsb, ns_o):
    nhv = nh[...]
    agg = jnp.where(pl.program_id(0) < 5, p0[...], p1[...])
    h = jnp.dot(nhv, nw1a[...], preferred_element_type=jnp.float32)
    h = h + jnp.dot(agg, nw1b[...], preferred_element_type=jnp.float32)
    h = jnp.maximum(h + nb1[...], 0.0)
    nh2 = jnp.dot(h, nw2[...], preferred_element_type=jnp.float32) + nb2[...]
    ns_o[...] = jax.nn.sigmoid(
        jnp.dot(nh2, nsw[...], preferred_element_type=jnp.float32) + nsb[...])


def _node_final_call(nh, p0, p1, nw1a, nw1b, nb1, nw2, nb2, nsw, nsb):
    nblk = pl.BlockSpec((BNP, D), lambda i: (i, 0))
    loblk = pl.BlockSpec((BNP, D), lambda i: (jnp.minimum(i, 4), 0))
    hiblk = pl.BlockSpec((BNP, D), lambda i: (jnp.maximum(i, 5) - 5, 0))
    wblk = pl.BlockSpec((D, D), lambda i: (0, 0))
    vblk = pl.BlockSpec((1, D), lambda i: (0, 0))
    return pl.pallas_call(
        _node_final_body,
        grid=(N_PAD // BNP,),
        in_specs=[nblk, loblk, hiblk, wblk, wblk, vblk, wblk, vblk,
                  pl.BlockSpec((D, 1), lambda i: (0, 0)),
                  pl.BlockSpec((1, 1), lambda i: (0, 0))],
        out_specs=pl.BlockSpec((BNP, 1), lambda i: (i, 0)),
        out_shape=jax.ShapeDtypeStruct((N_PAD, 1), jnp.float32),
        compiler_params=pltpu.CompilerParams(
            dimension_semantics=("parallel",)),
    )(nh, p0, p1, nw1a, nw1b, nb1, nw2, nb2, nsw, nsb)


@jax.jit
def kernel(node_embedding, relation_embedding, question_embedding, edge_index,
           edge_type, q_w, q_b, es_w, es_b, ns_w, ns_b,
           edge_w1_0, edge_b1_0, edge_w2_0, edge_b2_0,
           node_w1_0, node_b1_0, node_w2_0, node_b2_0,
           edge_w1_1, edge_b1_1, edge_w2_1, edge_b2_1,
           node_w1_1, node_b1_1, node_w2_1, node_b2_1):
    del edge_type
    src = edge_index[0]
    dst = edge_index[1]
    pad = jnp.zeros((E_PAD - N_EDGES,), jnp.int32)
    srcp = jnp.concatenate([src, pad])
    dstp = jnp.concatenate([dst, pad])
    zeros_acc = jnp.zeros((S_ACC, D), jnp.float32)
    dump_row = N_HALF + (jnp.arange(N_EDGES, dtype=jnp.int32) & 127)
    srcl = jnp.where(src < N_HALF, src, dump_row)
    dstl = jnp.where(dst < N_HALF, dst, dump_row)
    srch = jnp.where(src >= N_HALF, src - N_HALF, dump_row)
    dsth = jnp.where(dst >= N_HALF, dst - N_HALF, dump_row)

    def split4(w):
        return w[0:D], w[D:2 * D], w[2 * D:3 * D], w[3 * D:4 * D]

    w1s0, w1d0, w1e0, w1q0 = split4(edge_w1_0)
    w1s1, w1d1, w1e1, w1q1 = split4(edge_w1_1)
    nw1a0, nw1b0 = node_w1_0[0:D], node_w1_0[D:2 * D]
    nw1a1, nw1b1 = node_w1_1[0:D], node_w1_1[D:2 * D]

    r2 = lambda v: v.reshape(1, -1)

    qb0, qb1 = _q_call(question_embedding, q_w, r2(q_b),
                       w1q0, r2(edge_b1_0), w1q1, r2(edge_b1_1))

    a0, b0 = _prep_call(node_embedding, w1s0, w1d0)
    ga0, gb0 = _gather_call(a0, b0, srcp, dstp)
    ueh0 = _edge_call(ga0, gb0, relation_embedding,
                      w1e0, edge_w2_0, qb0, r2(edge_b2_0))
    p0, p1 = _scatter_call(ueh0, srcl, dstl, srch, dsth, zeros_acc)
    nh1, a1, b1 = _node_prep_call(node_embedding, p0, p1, nw1a0, nw1b0,
                                  r2(node_b1_0), node_w2_0, r2(node_b2_0),
                                  w1s1, w1d1)

    ga1, gb1 = _gather_call(a1, b1, srcp, dstp)
    ueh1, es = _edge_score_call(ga1, gb1, ueh0,
                                w1e1, edge_w2_1,
                                qb1, r2(edge_b2_1), es_w, r2(es_b).T)
    q0, q1 = _scatter_call(ueh1, srcl, dstl, srch, dsth, zeros_acc)
    ns = _node_final_call(nh1, q0, q1, nw1a1, nw1b1,
                          r2(node_b1_1), node_w2_1, r2(node_b2_1),
                          ns_w, r2(ns_b).T)

    return (ns.reshape(N_PAD)[:N_NODES], es.reshape(N_EDGES))
